# baseline (speedup 1.0000x reference)
import functools

import jax
import jax.numpy as jnp
from jax import lax
from jax.experimental import pallas as pl
from jax.experimental.pallas import tpu as pltpu
from jax.experimental.pallas import tpu_sc as plsc

VOCAB = 1000000
EMB_DIM = 64

NC = 2
NS = 16
NW = NC * NS

TOTAL = 4096 * 200
B_PER_W = TOTAL // NW
IDX_ROW = 128
KB = 8
BLK = KB * IDX_ROW
NBLK = B_PER_W // BLK


def _gather_body(idx_hbm, table_hbm, out_hbm, idx_v, rows_v, sem_g):
    c = lax.axis_index("c")
    s = lax.axis_index("s")
    wid = s * NC + c

    def blk_body(blk, carry):
        pltpu.sync_copy(idx_hbm.at[wid, blk], idx_v)
        copies = []
        for j in range(KB):
            cp = pltpu.make_async_copy(
                table_hbm.at[idx_v.at[j]],
                rows_v.at[pl.ds(j * IDX_ROW, IDX_ROW)],
                sem_g,
            )
            cp.start()
            copies.append(cp)
        for cp in copies:
            cp.wait()
        base = wid * B_PER_W + blk * BLK
        pltpu.sync_copy(rows_v, out_hbm.at[pl.ds(base, BLK)])
        return carry

    lax.fori_loop(0, NBLK, blk_body, 0)


@functools.partial(jax.jit, donate_argnums=())
def _embed(idx, table):
    mesh = plsc.VectorSubcoreMesh(core_axis_name="c", subcore_axis_name="s")
    run = pl.kernel(
        _gather_body,
        out_type=jax.ShapeDtypeStruct((TOTAL, EMB_DIM), jnp.float32),
        mesh=mesh,
        scratch_types=[
            pltpu.VMEM((KB, IDX_ROW), jnp.int32),
            pltpu.VMEM((BLK, EMB_DIM), jnp.float32),
            pltpu.SemaphoreType.DMA,
        ],
        compiler_params=pltpu.CompilerParams(use_tc_tiling_on_sc=False),
    )
    return run(idx, table)


def kernel(inp, emb_weight):
    idx = inp.reshape(NW, NBLK, KB, IDX_ROW)
    out = _embed(idx, emb_weight)
    return out.reshape(inp.shape[0], inp.shape[1], EMB_DIM)


# speedup vs baseline: 1.0170x; 1.0170x over previous
import functools

import jax
import jax.numpy as jnp
from jax import lax
from jax.experimental import pallas as pl
from jax.experimental.pallas import tpu as pltpu
from jax.experimental.pallas import tpu_sc as plsc

VOCAB = 1000000
EMB_DIM = 64

NC = 2
NS = 16
NW = NC * NS

TOTAL = 4096 * 200
B_PER_W = TOTAL // NW
IDX_ROW = 128
NG = B_PER_W // IDX_ROW
NBUF = 8
NROUND = NG // NBUF


def _gather_body(idx_hbm, table_hbm, out_hbm, idx_v, rows_v, sem_g, sem_w):
    c = lax.axis_index("c")
    s = lax.axis_index("s")
    wid = s * NC + c
    out_base = wid * B_PER_W

    pltpu.sync_copy(idx_hbm.at[wid], idx_v)

    def gather(g, b):
        return pltpu.make_async_copy(
            table_hbm.at[idx_v.at[g]], rows_v.at[b], sem_g.at[b]
        )

    def writeback(g, b):
        return pltpu.make_async_copy(
            rows_v.at[b], out_hbm.at[pl.ds(out_base + g * IDX_ROW, IDX_ROW)], sem_w.at[b]
        )

    for b in range(NBUF):
        gather(b, b).start()

    def round_body(i, carry):
        for b in range(NBUF):
            g = i * NBUF + b
            gather(g, b).wait()
            writeback(g, b).start()

            @pl.when(i < NROUND - 1)
            def _():
                writeback(g, b).wait()
                gather(g + NBUF, b).start()

        return carry

    lax.fori_loop(0, NROUND, round_body, 0)

    for b in range(NBUF):
        writeback(0, b).wait()


@jax.jit
def _embed(idx, table):
    mesh = plsc.VectorSubcoreMesh(core_axis_name="c", subcore_axis_name="s")
    run = pl.kernel(
        _gather_body,
        out_type=jax.ShapeDtypeStruct((TOTAL, EMB_DIM), jnp.float32),
        mesh=mesh,
        scratch_types=[
            pltpu.VMEM((NG, IDX_ROW), jnp.int32),
            pltpu.VMEM((NBUF, IDX_ROW, EMB_DIM), jnp.float32),
            pltpu.SemaphoreType.DMA((NBUF,)),
            pltpu.SemaphoreType.DMA((NBUF,)),
        ],
        compiler_params=pltpu.CompilerParams(use_tc_tiling_on_sc=False),
    )
    return run(idx, table)


def kernel(inp, emb_weight):
    idx = inp.reshape(NW, NG, IDX_ROW)
    out = _embed(idx, emb_weight)
    return out.reshape(inp.shape[0], inp.shape[1], EMB_DIM)
